# Initial kernel scaffold; baseline (speedup 1.0000x reference)
import jax
import jax.numpy as jnp
from jax.experimental import pallas as pl


def kernel(x, edge_index, edge_attr, fragments, fragments_edge_index, batch, params):
    raise NotImplementedError("write your pallas kernel here")


# speedup vs baseline: 1.4602x; 1.4602x over previous
import functools

import jax
import jax.numpy as jnp
from jax import lax
from jax.experimental import pallas as pl
from jax.experimental.pallas import tpu as pltpu
from jax.experimental.pallas import tpu_sc as plsc

N = 10000
E = 320000
D_IN = 128
D_EDGE = 16
H = 128
NF = 2000
FE = 40000
NB = 64
FRAG_VOCAB = 20
OUT = 1

NC = 2
NS = 16
NW = NC * NS

CH = 128
E_PAD = 327680
PER_W_E = E_PAD // NW
NCH_E = PER_W_E // CH
FE_PAD = 40960
PER_W_F = FE_PAD // NW
NCH_F = PER_W_F // CH
N_PAD = 10240


def _sds(shape):
    return jax.ShapeDtypeStruct(shape, jnp.float32)


_MESH = plsc.VectorSubcoreMesh(core_axis_name="c", subcore_axis_name="s",
                               num_cores=NC, num_subcores=NS)

_RPT = N_PAD // NS
_NSTG = _RPT // CH


def _make_gather(nch, kb, nrow):
    def body(tab_hbm, idx_hbm, out_hbm, idx_i, rows_v, sem):
        c = lax.axis_index("c")
        s = lax.axis_index("s")
        wid = s * NC + c

        def _blk(b, bc):
            pltpu.sync_copy(idx_hbm.at[wid, pl.ds(b * kb, kb)], idx_i)

            def _chunk(j, cc):
                pos = (wid * nch + b * kb + j) * CH
                pltpu.async_copy(tab_hbm.at[idx_i.at[j]], rows_v, sem).wait()
                pltpu.sync_copy(rows_v, out_hbm.at[pl.ds(pos, CH)])
                return cc

            lax.fori_loop(0, kb, _chunk, 0)
            return bc

        lax.fori_loop(0, nch // kb, _blk, 0)

    return functools.partial(
        pl.kernel,
        out_type=_sds((NW * nch * CH, H)),
        mesh=_MESH,
        scratch_types=[
            pltpu.VMEM((kb, CH), jnp.int32),
            pltpu.VMEM((CH, H), jnp.float32),
            pltpu.SemaphoreType.DMA,
        ],
    )(body)


def _make_scatter(nch, kb, with_msg):
    def body(*refs):
        if with_msg:
            (msg_hbm, idx_hbm, init_hbm, out_hbm,
             idx_i, msg_v, acc_sh, sem) = refs
        else:
            (idx_hbm, init_hbm, out_hbm,
             idx_i, msg_v, acc_sh, sem) = refs
        c = lax.axis_index("c")
        s = lax.axis_index("s")
        wid = s * NC + c
        pltpu.sync_copy(init_hbm.at[0], msg_v)

        def _zb(t, tc):
            pltpu.sync_copy(msg_v, acc_sh.at[pl.ds(s * _RPT + t * CH, CH)])
            return tc

        lax.fori_loop(0, _NSTG, _zb, 0)
        if not with_msg:
            pltpu.sync_copy(init_hbm.at[1], msg_v)
        plsc.subcore_barrier()

        def _blk(b, bc):
            pltpu.sync_copy(idx_hbm.at[wid, pl.ds(b * kb, kb)], idx_i)

            def _chunk(j, cc):
                if with_msg:
                    pos = (wid * nch + b * kb + j) * CH
                    pltpu.sync_copy(msg_hbm.at[pl.ds(pos, CH)], msg_v)
                pltpu.sync_copy(msg_v, acc_sh.at[idx_i.at[j]], add=True)
                return cc

            lax.fori_loop(0, kb, _chunk, 0)
            return bc

        lax.fori_loop(0, nch // kb, _blk, 0)
        plsc.subcore_barrier()

        def _wb(t, tc):
            sl = pl.ds(s * _RPT + t * CH, CH)
            pltpu.sync_copy(acc_sh.at[sl], msg_v)
            pltpu.sync_copy(msg_v, out_hbm.at[c, sl])
            return tc

        lax.fori_loop(0, _NSTG, _wb, 0)

    return functools.partial(
        pl.kernel,
        out_type=_sds((NC, N_PAD, H)),
        mesh=_MESH,
        scratch_types=[
            pltpu.VMEM((kb, CH), jnp.int32),
            pltpu.VMEM((CH, H), jnp.float32),
            pltpu.VMEM_SHARED((N_PAD, H), jnp.float32),
            pltpu.SemaphoreType.DMA,
        ],
    )(body)


_gather_e = _make_gather(NCH_E, 8, N)
_gather_f = _make_gather(NCH_F, 10, NF)
_scatter_e = _make_scatter(NCH_E, 8, True)
_scatter_f = _make_scatter(NCH_F, 10, True)
_count_f = _make_scatter(NCH_F, 10, False)


def _encode_body(x_ref, aw_ref, ab_ref, frag_ref, femb_ref, h0_ref, xfrag_ref):
    h0_ref[...] = (
        jnp.dot(x_ref[...], aw_ref[...], preferred_element_type=jnp.float32,
                precision=lax.Precision.HIGHEST)
        + ab_ref[...]
    )
    oh = (frag_ref[...]
          == lax.broadcasted_iota(jnp.int32, (NF, FRAG_VOCAB), 1)
          ).astype(jnp.float32)
    xfrag_ref[...] = jnp.dot(oh, femb_ref[...],
                             preferred_element_type=jnp.float32,
                precision=lax.Precision.HIGHEST)


_encode = pl.pallas_call(
    _encode_body,
    out_shape=(_sds((N, H)), _sds((NF, H))),
)


def _embed_body(ea_ref, w1_ref, b1_ref, w2_ref, b2_ref, e1_ref, e2_ref):
    ea = ea_ref[...]
    e1_ref[...] = (
        jnp.dot(ea, w1_ref[...], preferred_element_type=jnp.float32,
                precision=lax.Precision.HIGHEST)
        + b1_ref[...]
    )
    e2_ref[...] = (
        jnp.dot(ea, w2_ref[...], preferred_element_type=jnp.float32,
                precision=lax.Precision.HIGHEST)
        + b2_ref[...]
    )


_EBLK = 4096

_edge_embed = pl.pallas_call(
    _embed_body,
    grid=(E_PAD // _EBLK,),
    in_specs=[
        pl.BlockSpec((_EBLK, D_EDGE), lambda i: (i, 0)),
        pl.BlockSpec((D_EDGE, H), lambda i: (0, 0)),
        pl.BlockSpec((1, H), lambda i: (0, 0)),
        pl.BlockSpec((D_EDGE, H), lambda i: (0, 0)),
        pl.BlockSpec((1, H), lambda i: (0, 0)),
    ],
    out_specs=(
        pl.BlockSpec((_EBLK, H), lambda i: (i, 0)),
        pl.BlockSpec((_EBLK, H), lambda i: (i, 0)),
    ),
    out_shape=(_sds((E_PAD, H)), _sds((E_PAD, H))),
)


def _relu_add_body(a_ref, b_ref, o_ref):
    o_ref[...] = jnp.maximum(a_ref[...] + b_ref[...], 0.0)


_relu_add = pl.pallas_call(
    _relu_add_body,
    grid=(E_PAD // _EBLK,),
    in_specs=[
        pl.BlockSpec((_EBLK, H), lambda i: (i, 0)),
        pl.BlockSpec((_EBLK, H), lambda i: (i, 0)),
    ],
    out_specs=pl.BlockSpec((_EBLK, H), lambda i: (i, 0)),
    out_shape=_sds((E_PAD, H)),
)


def _frag_combine_body(h0_ref, s0_ref, s1_ref, c0_ref, c1_ref, h_ref):
    cnt = c0_ref[...][:, 0:1] + c1_ref[...][:, 0:1]
    h_ref[...] = h0_ref[...] + (s0_ref[...] + s1_ref[...]) / jnp.maximum(cnt, 1.0)


_frag_combine = pl.pallas_call(
    _frag_combine_body,
    out_shape=_sds((N, H)),
)


def _bn(z, g, b):
    m = jnp.mean(z, axis=0, keepdims=True)
    v = jnp.mean((z - m) ** 2, axis=0, keepdims=True)
    return (z - m) / jnp.sqrt(v + 1e-5) * g + b


def _layer_mlp_body(h_ref, a0_ref, a1_ref, eps_ref, w1_ref, b1_ref, g1_ref,
                    be1_ref, w2_ref, b2_ref, bg_ref, bb_ref, out_ref):
    z = (1.0 + eps_ref[...]) * h_ref[...] + a0_ref[...] + a1_ref[...]
    z = jnp.dot(z, w1_ref[...], preferred_element_type=jnp.float32,
                precision=lax.Precision.HIGHEST) + b1_ref[...]
    z = jnp.maximum(_bn(z, g1_ref[...], be1_ref[...]), 0.0)
    z = jnp.dot(z, w2_ref[...], preferred_element_type=jnp.float32,
                precision=lax.Precision.HIGHEST) + b2_ref[...]
    out_ref[...] = jnp.maximum(_bn(z, bg_ref[...], bb_ref[...]), 0.0)


_layer_mlp = pl.pallas_call(
    _layer_mlp_body,
    out_shape=_sds((N, H)),
)


def _final_body(h_ref, aw1_ref, ab1_ref, aw2_ref, ab2_ref, batch_ref,
                ow1_ref, ob1_ref, ow2_ref, ob2_ref, out_ref):
    t = jnp.maximum(
        jnp.dot(h_ref[...], aw1_ref[...], preferred_element_type=jnp.float32,
                precision=lax.Precision.HIGHEST)
        + ab1_ref[...], 0.0)
    t = jnp.maximum(
        jnp.dot(t, aw2_ref[...], preferred_element_type=jnp.float32,
                precision=lax.Precision.HIGHEST)
        + ab2_ref[...], 0.0)
    oh = (batch_ref[...]
          == lax.broadcasted_iota(jnp.int32, (N, NB), 1)).astype(jnp.float32)
    gs = lax.dot_general(oh, t, (((0,), (0,)), ((), ())),
                         preferred_element_type=jnp.float32,
                precision=lax.Precision.HIGHEST)
    gc = jnp.sum(oh, axis=0)[:, None]
    g = gs / jnp.maximum(gc, 1.0)
    g = jnp.maximum(
        jnp.dot(g, ow1_ref[...], preferred_element_type=jnp.float32,
                precision=lax.Precision.HIGHEST)
        + ob1_ref[...], 0.0)
    o = jnp.dot(g, ow2_ref[...], preferred_element_type=jnp.float32,
                precision=lax.Precision.HIGHEST) + ob2_ref[...]
    out_ref[...] = jnp.broadcast_to(o, (NB, H))


_final = pl.pallas_call(
    _final_body,
    out_shape=_sds((NB, H)),
)


def kernel(x, edge_index, edge_attr, fragments, fragments_edge_index, batch,
           params):
    p = params
    l0, l1 = p["layers"][0], p["layers"][1]

    src = jnp.pad(edge_index[0], (0, E_PAD - E)).reshape(NW, NCH_E, CH)
    dst = jnp.pad(edge_index[1], (0, E_PAD - E),
                  constant_values=N).reshape(NW, NCH_E, CH)
    eap = jnp.pad(edge_attr, ((0, E_PAD - E), (0, 0)))
    col = jnp.pad(fragments_edge_index[1],
                  (0, FE_PAD - FE)).reshape(NW, NCH_F, CH)
    row = jnp.pad(fragments_edge_index[0], (0, FE_PAD - FE),
                  constant_values=N).reshape(NW, NCH_F, CH)

    h0, xfrag = _encode(x, p["atom_W"], p["atom_b"][None], fragments[:, None],
                        p["frag_emb"])
    e1, e2 = _edge_embed(eap, l0["bond_W"], l0["bond_b"][None],
                         l1["bond_W"], l1["bond_b"][None])
    init = jnp.concatenate([jnp.zeros((1, CH, H), jnp.float32),
                            jnp.ones((1, CH, H), jnp.float32)], axis=0)
    xg = _gather_f(xfrag, col)
    sfrag = _scatter_f(xg, row, init)
    cnt = _count_f(row, init)
    h = _frag_combine(h0, sfrag[0, :N], sfrag[1, :N], cnt[0, :N], cnt[1, :N])

    for lp, e in ((l0, e1), (l1, e2)):
        hs = _gather_e(h, src)
        msg = _relu_add(hs, e)
        agg = _scatter_e(msg, dst, init)
        h = _layer_mlp(h, agg[0, :N], agg[1, :N], lp["eps"][None, None],
                       lp["nn_W1"], lp["nn_b1"][None], lp["nn_g1"][None],
                       lp["nn_be1"][None], lp["nn_W2"], lp["nn_b2"][None],
                       lp["bn_g"][None], lp["bn_b"][None])

    ao, o = p["atom_out"], p["out"]
    fin = _final(h, ao["W1"], ao["b1"][None], ao["W2"], ao["b2"][None],
                 batch[:, None], o["W1"], o["b1"][None], o["W2"],
                 o["b2"][None])
    return fin[:, :OUT]
